# baseline (speedup 1.0000x reference)
import jax
import jax.numpy as jnp
from jax.experimental import pallas as pl

N = 100000
BL = 51200


def _mlp_kernel(x_ref, w1_ref, b1_ref, ws_ref, wi_ref, bs_ref, bi_ref,
                wc_ref, bc_ref, we_ref, be_ref, seg_ref, emb_ref, mar_ref):
    xb = x_ref[...]
    b1 = b1_ref[...].T
    h = jnp.maximum(
        jax.lax.dot_general(w1_ref[...], xb, (((0,), (0,)), ((), ())),
                            preferred_element_type=jnp.float32)
        + b1, 0.0)
    w2 = jnp.concatenate([ws_ref[...], wi_ref[...]], axis=0)
    b2 = jnp.concatenate([bs_ref[...].T, bi_ref[...].T], axis=0)
    g = jnp.maximum(
        jnp.dot(w2, h, preferred_element_type=jnp.float32) + b2, 0.0)
    z516 = jnp.zeros((5, 16), jnp.float32)
    z316 = jnp.zeros((3, 16), jnp.float32)
    wh = jnp.concatenate([
        jnp.concatenate([wc_ref[...], z516], axis=1),
        jnp.concatenate([z316, z316], axis=1),
        jnp.concatenate([z316, we_ref[:3]], axis=1),
        jnp.concatenate([z516, z516], axis=1),
        jnp.concatenate([jnp.zeros((1, 16), jnp.float32),
                         we_ref[3:4]], axis=1)], axis=0)
    bh = jnp.concatenate([
        bc_ref[...].T, jnp.zeros((3, 1), jnp.float32),
        be_ref[...].T[:3], jnp.zeros((5, 1), jnp.float32),
        be_ref[...].T[3:4]], axis=0)
    e17 = (jnp.dot(wh, g, preferred_element_type=jnp.float32)
           + bh)
    seg_ref[...] = e17[:5]
    emb_ref[...] = e17[8:11]
    mar_ref[...] = e17[16:17]


def kernel(x, frag_ids, group_ids, edge_index1, edge_index2, params):
    p = params
    xt = x.T
    w1 = p["Wb"]
    ws = p["Ws"].T
    wi = p["Wi"].T
    wc = p["Wcls"].T
    we = p["Wemb"].T
    b1 = p["bb"][None, :]
    bs = p["bs"][None, :]
    bi = p["bi"][None, :]
    bc = p["bcls"][None, :]
    be = p["bemb"][None, :]

    def lanes(i):
        return (0, i)

    def whole(i):
        return (0, 0)

    nblk = (N + BL - 1) // BL
    segt, embt, mart = pl.pallas_call(
        _mlp_kernel,
        grid=(nblk,),
        in_specs=[pl.BlockSpec((5, BL), lanes),
                  pl.BlockSpec(w1.shape, whole), pl.BlockSpec(b1.shape, whole),
                  pl.BlockSpec(ws.shape, whole), pl.BlockSpec(wi.shape, whole),
                  pl.BlockSpec(bs.shape, whole), pl.BlockSpec(bi.shape, whole),
                  pl.BlockSpec(wc.shape, whole), pl.BlockSpec(bc.shape, whole),
                  pl.BlockSpec(we.shape, whole), pl.BlockSpec(be.shape, whole)],
        out_specs=[pl.BlockSpec((5, BL), lanes),
                   pl.BlockSpec((3, BL), lanes),
                   pl.BlockSpec((1, BL), lanes)],
        out_shape=[jax.ShapeDtypeStruct((5, N), jnp.float32),
                   jax.ShapeDtypeStruct((3, N), jnp.float32),
                   jax.ShapeDtypeStruct((1, N), jnp.float32)],
    )(xt, w1, b1, ws, wi, bs, bi, wc, bc, we, be)
    return (segt.T, embt.T, mart.T)


# speedup vs baseline: 1.0402x; 1.0402x over previous
import jax
import jax.numpy as jnp
from jax.experimental import pallas as pl

N = 100000
BL = 51200


def _mlp_kernel(x_ref, w1_ref, b1_ref, ws_ref, wi_ref, bs_ref, bi_ref,
                wc_ref, bc_ref, we_ref, be_ref, seg_ref, emb_ref, mar_ref):
    xb = x_ref[...]
    b1 = b1_ref[...].T
    h = jnp.maximum(
        jax.lax.dot_general(w1_ref[...], xb, (((0,), (0,)), ((), ())),
                            preferred_element_type=jnp.float32)
        + b1, 0.0)
    w2 = jnp.concatenate([ws_ref[...], wi_ref[...]], axis=0)
    b2 = jnp.concatenate([bs_ref[...].T, bi_ref[...].T], axis=0)
    g = jnp.maximum(
        jnp.dot(w2, h, preferred_element_type=jnp.float32) + b2, 0.0)
    seg_ref[...] = (
        jnp.dot(wc_ref[...], g[:16], preferred_element_type=jnp.float32)
        + bc_ref[...].T)
    e4 = (jnp.dot(we_ref[...], g[16:32], preferred_element_type=jnp.float32)
          + be_ref[...].T)
    emb_ref[...] = e4[:3]
    mar_ref[...] = e4[3:4]


def kernel(x, frag_ids, group_ids, edge_index1, edge_index2, params):
    p = params
    xt = x.T
    w1 = p["Wb"]
    ws = p["Ws"].T
    wi = p["Wi"].T
    wc = p["Wcls"].T
    we = p["Wemb"].T
    b1 = p["bb"][None, :]
    bs = p["bs"][None, :]
    bi = p["bi"][None, :]
    bc = p["bcls"][None, :]
    be = p["bemb"][None, :]

    def lanes(i):
        return (0, i)

    def whole(i):
        return (0, 0)

    nblk = (N + BL - 1) // BL
    segt, embt, mart = pl.pallas_call(
        _mlp_kernel,
        grid=(nblk,),
        in_specs=[pl.BlockSpec((5, BL), lanes),
                  pl.BlockSpec(w1.shape, whole), pl.BlockSpec(b1.shape, whole),
                  pl.BlockSpec(ws.shape, whole), pl.BlockSpec(wi.shape, whole),
                  pl.BlockSpec(bs.shape, whole), pl.BlockSpec(bi.shape, whole),
                  pl.BlockSpec(wc.shape, whole), pl.BlockSpec(bc.shape, whole),
                  pl.BlockSpec(we.shape, whole), pl.BlockSpec(be.shape, whole)],
        out_specs=[pl.BlockSpec((5, BL), lanes),
                   pl.BlockSpec((3, BL), lanes),
                   pl.BlockSpec((1, BL), lanes)],
        out_shape=[jax.ShapeDtypeStruct((5, N), jnp.float32),
                   jax.ShapeDtypeStruct((3, N), jnp.float32),
                   jax.ShapeDtypeStruct((1, N), jnp.float32)],
    )(xt, w1, b1, ws, wi, bs, bi, wc, bc, we, be)
    return (segt.T, embt.T, mart.T)
